# Initial kernel scaffold; baseline (speedup 1.0000x reference)
import jax
import jax.numpy as jnp
from jax.experimental import pallas as pl


def kernel(hidden_states, pos_keys, Wi, bi, Wq, bq, Wk1, bk1, Wk2, bk2, Wsa, bsa, Wg1, bg1, Wg2, bg2, Wo, bo, ln_g, ln_b):
    raise NotImplementedError("write your pallas kernel here")


# speedup vs baseline: 11.4927x; 11.4927x over previous
import functools

import jax
import jax.numpy as jnp
import numpy as np
from jax.experimental import pallas as pl
from jax.experimental.pallas import tpu as pltpu

DECAY = 0.9
EPS_LN = 1e-5
FP = 384


def _dft_mats(D):
    F = D // 2 + 1
    j = np.arange(D, dtype=np.float64)[:, None]
    f = np.arange(FP, dtype=np.float64)[None, :]
    ang = 2.0 * np.pi * j * f / D
    valid = f < F
    cos = np.where(valid, np.cos(ang), 0.0).astype(np.float32)
    sin = np.where(valid, np.sin(ang), 0.0).astype(np.float32)
    fc = np.arange(FP, dtype=np.float64)[:, None]
    dg = np.arange(D, dtype=np.float64)[None, :]
    ang2 = 2.0 * np.pi * fc * dg / D
    w = np.where((fc == 0) | (fc == D // 2), 1.0, 2.0) * (fc < F)
    icr = (w * np.cos(ang2) / D).astype(np.float32)
    ici = (-w * np.sin(ang2) / D).astype(np.float32)
    wpar = ((np.where((f == 0) | (f == D // 2), 1.0, 2.0) * valid) / D
            ).astype(np.float32)
    return cos, sin, icr, ici, wpar


def _gelu(x):
    erf_part = 0.5 * x * (1.0 + jax.lax.erf(x * 0.7071067811865476))
    z = -x * 0.7071067811865476
    t = 1.0 / (1.0 + 0.3275911 * z)
    poly = t * (0.254829592 + t * (-0.284496736 + t * (
        1.421413741 + t * (-1.453152027 + t * 1.061405429))))
    tail = 0.5 * x * jnp.exp(-z * z) * poly
    return jnp.where(x < -2.0, tail, erf_part)


def _precompute_body(hs_ref, pos_ref, wi_ref, bi_ref, wq_ref, bq_ref,
                     wk1_ref, bk1_ref, wk2_ref, bk2_ref, wg1q_ref, wg1i_ref,
                     bg1_ref, woq_ref, bo_ref, cos_ref, sin_ref,
                     icr_ref, ici_ref,
                     g0_ref, o0_ref, bnd_ref, kr_ref, ki_ref):
    ts, b, h = hs_ref.shape
    x = hs_ref[...].reshape(ts * b, h)
    dot = functools.partial(jnp.dot, preferred_element_type=jnp.float32,
                            precision=jax.lax.Precision.DEFAULT)
    doth = functools.partial(jnp.dot, preferred_element_type=jnp.float32,
                             precision=jax.lax.Precision.HIGHEST)
    items = dot(x, wi_ref[...]) + bi_ref[...]
    queries = dot(x, wq_ref[...]) + bq_ref[...]
    kh = _gelu(dot(queries, wk1_ref[...]) + bk1_ref[...])
    keys = dot(kh, wk2_ref[...]) + bk2_ref[...]
    nrm = jnp.sqrt(jnp.sum(keys * keys, axis=-1, keepdims=True))
    keys = keys / jnp.maximum(nrm, 1e-12)
    cos = cos_ref[...]
    sin = sin_ref[...]
    kr_ref[...] = doth(keys, cos).reshape(ts, b, FP)
    ki_ref[...] = doth(keys, sin).reshape(ts, b, FP)
    ir = doth(items, cos).reshape(ts, b, FP)
    ii = -doth(items, sin).reshape(ts, b, FP)
    pos = pos_ref[...]
    pr = doth(pos, cos)[:, None, :]
    pi = -doth(pos, sin)[:, None, :]
    br = (ir * pr - ii * pi).reshape(ts * b, FP)
    bim = (ir * pi + ii * pr).reshape(ts * b, FP)
    d = icr_ref.shape[1]
    bnd_ref[...] = (doth(br, icr_ref[...])
                    + doth(bim, ici_ref[...])).reshape(ts, b, d)
    g0_ref[...] = (dot(queries, wg1q_ref[...]) + dot(items, wg1i_ref[...])
                   + bg1_ref[...]).reshape(ts, b, h)
    o0_ref[...] = (dot(queries, woq_ref[...]) + bo_ref[...]).reshape(ts, b, h)


def _bf(x):
    return x.astype(jnp.bfloat16).astype(jnp.float32)


def _scan_body(nblk, ts,
               g0_ref, o0_ref, hs_ref, bnd_ref, kr_ref, ki_ref,
               wg1s_ref, wg2_ref, bg2_ref, woc_ref,
               cos_ref, sin_ref, icr_ref, ici_ref,
               wsa_ref, lng_ref, lnb_ref,
               out_ref, hcm_ref,
               hcm_s):
    blk = pl.program_id(0)
    dot = functools.partial(jnp.dot, preferred_element_type=jnp.float32,
                            precision=jax.lax.Precision.DEFAULT)
    doth = functools.partial(jnp.dot, preferred_element_type=jnp.float32,
                             precision=jax.lax.Precision.HIGHEST)

    @pl.when(blk == 0)
    def _init():
        hcm_s[...] = jnp.zeros_like(hcm_s)

    b, n, d = hcm_s.shape
    wsab = _bf(wsa_ref[...])[None]
    wg1sb = _bf(wg1s_ref[...])
    wg2b = _bf(wg2_ref[...])
    cosm = cos_ref[...]
    sinm = sin_ref[...]
    icrm = icr_ref[...]
    icim = ici_ref[...]

    def step(j, carry):
        kr = kr_ref[j]
        ki = ki_ref[j]
        hcm = hcm_s[...]
        hflat = hcm.reshape(b * n, d)
        hr = doth(hflat, cosm).reshape(b, n, FP)
        hi = -doth(hflat, sinm).reshape(b, n, FP)
        rr = (hr * kr[:, None, :] - hi * ki[:, None, :]).reshape(b * n, FP)
        ri = (hr * ki[:, None, :] + hi * kr[:, None, :]).reshape(b * n, FP)
        ret = _bf(doth(rr, icrm) + doth(ri, icim)).reshape(b, n, d)
        scores = jnp.sum(ret * wsab, axis=-1)
        m = jnp.max(scores, axis=-1, keepdims=True)
        e = jnp.exp(scores - m)
        wts = e / jnp.sum(e, axis=-1, keepdims=True)
        ctx = jnp.sum(ret * _bf(wts)[:, :, None], axis=1)
        out_lin = o0_ref[j] + dot(ctx, woc_ref[...])
        stats = jnp.sqrt(jnp.sum(hcm * hcm, axis=-1))
        gpre = g0_ref[j] + dot(_bf(stats), wg1sb)
        gates = jax.nn.sigmoid(dot(_bf(_gelu(gpre)), wg2b) + bg2_ref[...])
        hcm_s[...] = DECAY * hcm + gates[:, :, None] * bnd_ref[j][:, None, :]
        x = hs_ref[j] + out_lin
        mu = jnp.mean(x, axis=-1, keepdims=True)
        xc = x - mu
        var = jnp.mean(xc * xc, axis=-1, keepdims=True)
        out_ref[j] = (xc * jax.lax.rsqrt(var + EPS_LN) * lng_ref[...]
                      + lnb_ref[...])
        return carry

    jax.lax.fori_loop(0, ts, step, 0)

    @pl.when(blk == nblk - 1)
    def _finish():
        hcm_ref[...] = hcm_s[...]


def kernel(hidden_states, pos_keys, Wi, bi, Wq, bq, Wk1, bk1, Wk2, bk2,
           Wsa, bsa, Wg1, bg1, Wg2, bg2, Wo, bo, ln_g, ln_b):
    B, S, H = hidden_states.shape
    D = Wi.shape[1]
    N = Wg2.shape[1]
    del bsa
    cos, sin, icr, ici, wpar = _dft_mats(D)

    TS1 = min(32, S) if S % 32 == 0 else min(16, S)
    TS2 = min(16, S)
    n1 = S // TS1
    n2 = S // TS2

    hs_sb = jnp.transpose(hidden_states, (1, 0, 2))
    row = lambda v: v.reshape(1, -1)
    f32 = jnp.float32

    full = lambda a: pl.BlockSpec(a.shape, lambda s: (0,) * a.ndim)
    cparams1 = pltpu.CompilerParams(
        dimension_semantics=("parallel",),
        vmem_limit_bytes=52 * 1024 * 1024)
    g0, o0, bnd, kr, ki = pl.pallas_call(
        _precompute_body,
        grid=(n1,),
        in_specs=[
            pl.BlockSpec((TS1, B, H), lambda s: (s, 0, 0)),
            pl.BlockSpec((TS1, D), lambda s: (s, 0)),
            full(Wi), pl.BlockSpec((1, D), lambda s: (0, 0)),
            full(Wq), pl.BlockSpec((1, H), lambda s: (0, 0)),
            full(Wk1), pl.BlockSpec((1, 2 * D), lambda s: (0, 0)),
            full(Wk2), pl.BlockSpec((1, D), lambda s: (0, 0)),
            pl.BlockSpec((H, H), lambda s: (0, 0)),
            pl.BlockSpec((D, H), lambda s: (0, 0)),
            pl.BlockSpec((1, H), lambda s: (0, 0)),
            pl.BlockSpec((H, H), lambda s: (0, 0)),
            pl.BlockSpec((1, H), lambda s: (0, 0)),
            pl.BlockSpec((D, FP), lambda s: (0, 0)),
            pl.BlockSpec((D, FP), lambda s: (0, 0)),
            pl.BlockSpec((FP, D), lambda s: (0, 0)),
            pl.BlockSpec((FP, D), lambda s: (0, 0)),
        ],
        out_specs=[
            pl.BlockSpec((TS1, B, H), lambda s: (s, 0, 0)),
            pl.BlockSpec((TS1, B, H), lambda s: (s, 0, 0)),
            pl.BlockSpec((TS1, B, D), lambda s: (s, 0, 0)),
            pl.BlockSpec((TS1, B, FP), lambda s: (s, 0, 0)),
            pl.BlockSpec((TS1, B, FP), lambda s: (s, 0, 0)),
        ],
        out_shape=[
            jax.ShapeDtypeStruct((S, B, H), f32),
            jax.ShapeDtypeStruct((S, B, H), f32),
            jax.ShapeDtypeStruct((S, B, D), f32),
            jax.ShapeDtypeStruct((S, B, FP), f32),
            jax.ShapeDtypeStruct((S, B, FP), f32),
        ],
        compiler_params=cparams1,
    )(hs_sb, pos_keys, Wi, row(bi), Wq, row(bq), Wk1, row(bk1), Wk2, row(bk2),
      Wg1[:H], Wg1[H:H + D], row(bg1), Wo[:H], row(bo),
      jnp.asarray(cos), jnp.asarray(sin), jnp.asarray(icr), jnp.asarray(ici))

    cparams2 = pltpu.CompilerParams(
        dimension_semantics=("arbitrary",),
        vmem_limit_bytes=52 * 1024 * 1024)
    stream = pl.BlockSpec((TS2, B, H), lambda s: (s, 0, 0))
    streamf = pl.BlockSpec((TS2, B, FP), lambda s: (s, 0, 0))
    out_sb, hcm = pl.pallas_call(
        functools.partial(_scan_body, n2, TS2),
        grid=(n2,),
        in_specs=[
            stream, stream, stream,
            pl.BlockSpec((TS2, B, D), lambda s: (s, 0, 0)),
            streamf, streamf,
            pl.BlockSpec((N, H), lambda s: (0, 0)),
            full(Wg2),
            pl.BlockSpec((1, N), lambda s: (0, 0)),
            pl.BlockSpec((D, H), lambda s: (0, 0)),
            pl.BlockSpec((D, FP), lambda s: (0, 0)),
            pl.BlockSpec((D, FP), lambda s: (0, 0)),
            pl.BlockSpec((FP, D), lambda s: (0, 0)),
            pl.BlockSpec((FP, D), lambda s: (0, 0)),
            pl.BlockSpec((1, D), lambda s: (0, 0)),
            pl.BlockSpec((1, H), lambda s: (0, 0)),
            pl.BlockSpec((1, H), lambda s: (0, 0)),
        ],
        out_specs=[
            pl.BlockSpec((TS2, B, H), lambda s: (s, 0, 0)),
            pl.BlockSpec((B, N, D), lambda s: (0, 0, 0)),
        ],
        out_shape=[
            jax.ShapeDtypeStruct((S, B, H), f32),
            jax.ShapeDtypeStruct((B, N, D), f32),
        ],
        scratch_shapes=[
            pltpu.VMEM((B, N, D), f32),
        ],
        compiler_params=cparams2,
    )(g0, o0, hs_sb, bnd, kr, ki,
      Wg1[H + D:], Wg2, row(bg2), Wo[H:],
      jnp.asarray(cos), jnp.asarray(sin), jnp.asarray(icr), jnp.asarray(ici),
      Wsa.T, row(ln_g), row(ln_b))

    return jnp.transpose(out_sb, (1, 0, 2)), hcm
